# baseline (speedup 1.0000x reference)
import jax
import jax.numpy as jnp
from jax import lax
from jax.experimental import pallas as pl
from jax.experimental.pallas import tpu as pltpu
from jax.experimental.pallas import tpu_sc as plsc

N, E = 10000, 320000
NODE_IN, EDGE_IN, H = 128, 16, 128
HH = H // 2

NC, NS, LANES = 2, 16, 16
NW = NC * NS
EW = E // NW
C = 80
NCHUNK = EW // C
ST = 624
TAIL = N - NS * ST

_SQ12 = 0.7071067811865476


def _gelu_sc(v):
    z = v * _SQ12
    az = jnp.abs(z)
    t = 1.0 / (1.0 + 0.47047 * az)
    pe = (t * (0.3480242 + t * (-0.0958798 + t * 0.7478556))) * jnp.exp(-z * z)
    q = 0.5 * v * pe
    return jnp.where(v < 0, q, v - q)


def _gelu(v):
    z = v * _SQ12
    az = jnp.abs(z)
    t = 1.0 / (1.0 + 0.3275911 * az)
    poly = t * (0.254829592 + t * (-0.284496736 + t * (1.421413741
               + t * (-1.453152027 + t * 1.061405429))))
    erf_abs = 1.0 - poly * jnp.exp(-az * az)
    erf = jnp.where(z < 0, -erf_abs, erf_abs)
    return 0.5 * v * (1.0 + erf)


def _ln(v, g, b, eps=1e-5):
    m = v.mean(-1, keepdims=True)
    var = ((v - m) ** 2).mean(-1, keepdims=True)
    return (v - m) * lax.rsqrt(var + eps) * g + b


def _dot(a, b):
    return jax.lax.dot_general(a, b, (((1,), (0,)), ((), ())),
                               preferred_element_type=jnp.float32)


_NB = 2000
_NG = N // _NB
_EB = 4000
_EG = E // _EB


def _full(shape):
    return pl.BlockSpec(shape, lambda i: tuple(0 for _ in shape))


def _node_encode_body(x_ref, w_ref, b_ref, g_ref, bb_ref, wij_ref,
                      o_ref, *ab_refs):
    y = _dot(x_ref[...], w_ref[...]) + b_ref[...]
    h = _gelu(_ln(y, g_ref[...], bb_ref[...]))
    o_ref[...] = h
    ab = _dot(h, wij_ref[...])
    for q in range(4):
        ab_refs[q][...] = ab[:, q * HH:(q + 1) * HH]


def _edge_encode_body(ea_ref, w_ref, b_ref, g_ref, bb_ref,
                      wl_ref, bl_ref, olo_ref, ohi_ref):
    y = _dot(ea_ref[...], w_ref[...]) + b_ref[...]
    ea = _gelu(_ln(y, g_ref[...], bb_ref[...]))
    ec = _dot(ea, wl_ref[...]) + bl_ref[...]
    olo_ref[...] = ec[:, :HH]
    ohi_ref[...] = ec[:, HH:]


def _node_pre_body(h_ref, w_ref, *o_refs):
    y = _dot(h_ref[...], w_ref[...])
    for q in range(4):
        o_refs[q][...] = y[:, q * HH:(q + 1) * HH]


def _new_h(h_ref, s_ref, dp_ref, w2_ref, b2_ref, uh_ref, ua_ref, ub_ref,
           g_ref, bb_ref):
    s2 = jnp.concatenate([s_ref[0, 0] + s_ref[1, 0],
                          s_ref[0, 1] + s_ref[1, 1]], axis=-1)
    dsum = dp_ref[0] + dp_ref[1]
    aggr = _dot(s2, w2_ref[...]) + dsum[:, 0:1] * b2_ref[...]
    t = _dot(h_ref[...], uh_ref[...]) + _dot(aggr, ua_ref[...]) + ub_ref[...]
    return h_ref[...] + _ln(t, g_ref[...], bb_ref[...])


def _update_pre_body(h_ref, s_ref, dp_ref, w2_ref, b2_ref,
                     uh_ref, ua_ref, ub_ref, g_ref, bb_ref, wij_ref,
                     o_ref, *ab_refs):
    hn = _new_h(h_ref, s_ref, dp_ref, w2_ref, b2_ref, uh_ref, ua_ref,
                ub_ref, g_ref, bb_ref)
    o_ref[...] = hn
    ab = _dot(hn, wij_ref[...])
    for q in range(4):
        ab_refs[q][...] = ab[:, q * HH:(q + 1) * HH]


def _update_head_body(h_ref, s_ref, dp_ref, w2_ref, b2_ref,
                      uh_ref, ua_ref, ub_ref, g_ref, bb_ref,
                      w1_ref, b1_ref, hw2_ref, hb2_ref, w3_ref, b3_ref,
                      o_ref, acc_ref, mut_ref):
    i = pl.program_id(0)
    hn = _new_h(h_ref, s_ref, dp_ref, w2_ref, b2_ref, uh_ref, ua_ref,
                ub_ref, g_ref, bb_ref)

    @pl.when(i == 0)
    def _():
        acc_ref[...] = jnp.zeros_like(acc_ref)
        mut_ref[...] = hn[0:1, :]

    acc_ref[...] += jnp.sum(hn, axis=0, keepdims=True)

    @pl.when(i == _NG - 1)
    def _():
        glob = acc_ref[...] * (1.0 / N)
        r = jnp.concatenate([mut_ref[...], glob], axis=-1)
        r = _gelu(_dot(r, w1_ref[...]) + b1_ref[...])
        r = _gelu(_dot(r, hw2_ref[...]) + hb2_ref[...])
        o_ref[...] = _dot(r, w3_ref[...]) + b3_ref[...]


def _head_body(h_ref, w1_ref, b1_ref, w2_ref, b2_ref, w3_ref, b3_ref,
               o_ref, acc_ref, mut_ref):
    i = pl.program_id(0)

    @pl.when(i == 0)
    def _():
        acc_ref[...] = jnp.zeros_like(acc_ref)
        mut_ref[...] = h_ref[0:1, :]

    acc_ref[...] += jnp.sum(h_ref[...], axis=0, keepdims=True)

    @pl.when(i == _NG - 1)
    def _():
        glob = acc_ref[...] * (1.0 / N)
        r = jnp.concatenate([mut_ref[...], glob], axis=-1)
        r = _gelu(_dot(r, w1_ref[...]) + b1_ref[...])
        r = _gelu(_dot(r, w2_ref[...]) + b2_ref[...])
        o_ref[...] = _dot(r, w3_ref[...]) + b3_ref[...]


def _node_encode(x, wT, b, g, bb, wijT):
    return pl.pallas_call(
        _node_encode_body,
        grid=(_NG,),
        in_specs=[pl.BlockSpec((_NB, NODE_IN), lambda i: (i, 0)),
                  _full((NODE_IN, H)), _full((1, H)), _full((1, H)),
                  _full((1, H)), _full((H, 2 * H))],
        out_specs=[pl.BlockSpec((_NB, H), lambda i: (i, 0))]
                  + [pl.BlockSpec((_NB, HH), lambda i: (i, 0))] * 4,
        out_shape=[jax.ShapeDtypeStruct((N, H), jnp.float32)]
                  + [jax.ShapeDtypeStruct((N, HH), jnp.float32)] * 4,
    )(x, wT, b, g, bb, wijT)


def _edge_encode(ea, wT, b, g, bb, wlT, bl):
    specs = [pl.BlockSpec((_EB, EDGE_IN), lambda i: (i, 0)),
             _full((EDGE_IN, H)), _full((1, H)), _full((1, H)), _full((1, H)),
             _full((H, H)), _full((1, H))]
    o_spec = pl.BlockSpec((_EB, HH), lambda i: (i, 0))
    o_shape = jax.ShapeDtypeStruct((E, HH), jnp.float32)
    return pl.pallas_call(
        _edge_encode_body,
        grid=(_EG,),
        in_specs=specs,
        out_specs=[o_spec] * 2,
        out_shape=[o_shape] * 2,
    )(ea, wT, b, g, bb, wlT, bl)


def _node_pre(h, wT):
    return pl.pallas_call(
        _node_pre_body,
        grid=(_NG,),
        in_specs=[pl.BlockSpec((_NB, H), lambda i: (i, 0)), _full((H, 2 * H))],
        out_specs=[pl.BlockSpec((_NB, HH), lambda i: (i, 0))] * 4,
        out_shape=[jax.ShapeDtypeStruct((N, HH), jnp.float32)] * 4,
    )(h, wT)


_UPD_SPECS = [pl.BlockSpec((_NB, H), lambda i: (i, 0)),
              pl.BlockSpec((NC, 2, _NB, HH), lambda i: (0, 0, i, 0)),
              pl.BlockSpec((NC, _NB, LANES), lambda i: (0, i, 0)),
              _full((H, H)), _full((1, H)), _full((H, H)), _full((H, H)),
              _full((1, H)), _full((1, H)), _full((1, H))]


def _update_pre(h, s, dp, w2T, b2, uhT, uaT, ub, g, bb, wijT):
    return pl.pallas_call(
        _update_pre_body,
        grid=(_NG,),
        in_specs=_UPD_SPECS + [_full((H, 2 * H))],
        out_specs=[pl.BlockSpec((_NB, H), lambda i: (i, 0))]
                  + [pl.BlockSpec((_NB, HH), lambda i: (i, 0))] * 4,
        out_shape=[jax.ShapeDtypeStruct((N, H), jnp.float32)]
                  + [jax.ShapeDtypeStruct((N, HH), jnp.float32)] * 4,
    )(h, s, dp, w2T, b2, uhT, uaT, ub, g, bb, wijT)


def _update_head(h, s, dp, w2T, b2, uhT, uaT, ub, g, bb,
                 w1T, b1, hw2T, hb2, w3T, b3):
    return pl.pallas_call(
        _update_head_body,
        grid=(_NG,),
        in_specs=_UPD_SPECS + [_full((2 * H, H)), _full((1, H)),
                               _full((H, H // 2)), _full((1, H // 2)),
                               _full((H // 2, 1)), _full((1, 1))],
        out_specs=pl.BlockSpec((1, 1), lambda i: (0, 0)),
        out_shape=jax.ShapeDtypeStruct((1, 1), jnp.float32),
        scratch_shapes=[pltpu.VMEM((1, H), jnp.float32),
                        pltpu.VMEM((1, H), jnp.float32)],
    )(h, s, dp, w2T, b2, uhT, uaT, ub, g, bb,
      w1T, b1, hw2T, hb2, w3T, b3)


def _head(h, w1T, b1, w2T, b2, w3T, b3):
    return pl.pallas_call(
        _head_body,
        grid=(_NG,),
        in_specs=[pl.BlockSpec((_NB, H), lambda i: (i, 0)),
                  _full((2 * H, H)), _full((1, H)),
                  _full((H, H // 2)), _full((1, H // 2)),
                  _full((H // 2, 1)), _full((1, 1))],
        out_specs=pl.BlockSpec((1, 1), lambda i: (0, 0)),
        out_shape=jax.ShapeDtypeStruct((1, 1), jnp.float32),
        scratch_shapes=[pltpu.VMEM((1, H), jnp.float32),
                        pltpu.VMEM((1, H), jnp.float32)],
    )(h, w1T, b1, w2T, b2, w3T, b3)


def _mesh():
    return plsc.VectorSubcoreMesh(core_axis_name="c", subcore_axis_name="s",
                                  num_cores=NC, num_subcores=NS)


_CD = 2000
_NDC = EW // _CD


def _degree_body(dst_hbm, out_hbm, idx_v, ones_v, zb_v, deg_sh, sem):
    core = lax.axis_index("c")
    sub = lax.axis_index("s")
    wid = sub * NC + core
    r0 = pl.multiple_of(sub * ST, 8)

    def fill_z(r, carry):
        zb_v[r, :] = jnp.zeros((LANES,), jnp.float32)
        return carry
    lax.fori_loop(0, ST, fill_z, 0)

    def fill_o(r, carry):
        ones_v[r, :] = jnp.ones((LANES,), jnp.float32)
        return carry
    lax.fori_loop(0, _CD, fill_o, 0)

    pltpu.sync_copy(zb_v, deg_sh.at[pl.ds(r0, ST)])

    @pl.when(sub == NS - 1)
    def _():
        pltpu.sync_copy(zb_v.at[pl.ds(0, TAIL)],
                        deg_sh.at[pl.ds(NS * ST, TAIL)])

    plsc.subcore_barrier()

    def chunk(k, carry):
        base = pl.multiple_of(wid * EW + k * _CD, 8)
        pltpu.sync_copy(dst_hbm.at[pl.ds(base, _CD)], idx_v)
        pltpu.sync_copy(ones_v, deg_sh.at[idx_v], add=True)
        return carry
    lax.fori_loop(0, _NDC, chunk, 0)

    plsc.subcore_barrier()
    pltpu.sync_copy(deg_sh.at[pl.ds(r0, ST)], zb_v)
    pltpu.sync_copy(zb_v, out_hbm.at[core].at[pl.ds(r0, ST)])

    @pl.when(sub == NS - 1)
    def _():
        pltpu.sync_copy(deg_sh.at[pl.ds(NS * ST, TAIL)],
                        zb_v.at[pl.ds(0, TAIL)])
        pltpu.sync_copy(zb_v.at[pl.ds(0, TAIL)],
                        out_hbm.at[core].at[pl.ds(NS * ST, TAIL)])


def _degree(dst):
    k = pl.kernel(
        _degree_body,
        out_type=jax.ShapeDtypeStruct((NC, N, LANES), jnp.float32),
        mesh=_mesh(),
        compiler_params=pltpu.CompilerParams(use_tc_tiling_on_sc=False),
        scratch_types=[
            pltpu.VMEM((_CD,), jnp.int32),
            pltpu.VMEM((_CD, LANES), jnp.float32),
            pltpu.VMEM((ST, LANES), jnp.float32),
            pltpu.VMEM_SHARED((N, LANES), jnp.float32),
            pltpu.SemaphoreType.DMA,
        ],
    )
    return k(dst)


def _edge_pass_body(alo_hbm, blo_hbm, eclo_hbm, ahi_hbm, bhi_hbm, echi_hbm,
                    dsts_hbm, srcs_hbm, out_hbm,
                    dsts, srcs, ra0, rb0, re0, g0, ra1, rb1, re1, g1,
                    s_sh, sg0, sg1, ss0, ss1):
    core = lax.axis_index("c")
    sub = lax.axis_index("s")
    wid = sub * NC + core
    ras = (ra0, ra1)
    rbs = (rb0, rb1)
    res = (re0, re1)
    gs = (g0, g1)
    sgs = (sg0, sg1)
    sss = (ss0, ss1)
    r0 = pl.multiple_of(sub * ST, 8)

    pltpu.sync_copy(dsts_hbm.at[wid], dsts)
    pltpu.sync_copy(srcs_hbm.at[wid], srcs)

    for half, (a_hbm, b_hbm, ec_hbm) in enumerate(
            ((alo_hbm, blo_hbm, eclo_hbm), (ahi_hbm, bhi_hbm, echi_hbm))):
        def fill_z(r, carry):
            for j in range(HH // LANES):
                g0[r, pl.ds(j * LANES, LANES)] = jnp.zeros((LANES,),
                                                           jnp.float32)
            return carry
        lax.fori_loop(0, C, fill_z, 0)
        for t in range(ST // C):
            pltpu.sync_copy(g0, s_sh.at[pl.ds(r0 + t * C, C)])
        pltpu.sync_copy(g0.at[pl.ds(0, ST % C)],
                        s_sh.at[pl.ds(r0 + (ST // C) * C, ST % C)])

        @pl.when(sub == NS - 1)
        def _():
            pltpu.sync_copy(g0.at[pl.ds(0, TAIL)],
                            s_sh.at[pl.ds(NS * ST, TAIL)])

        plsc.subcore_barrier()

        def issue_gathers(kk, b):
            base = pl.multiple_of(wid * EW + kk * C, 8)
            pltpu.async_copy(a_hbm.at[dsts.at[kk]], ras[b], sgs[b])
            pltpu.async_copy(b_hbm.at[srcs.at[kk]], rbs[b], sgs[b])
            pltpu.async_copy(ec_hbm.at[pl.ds(base, C)], res[b], sgs[b])

        for b in range(2):
            issue_gathers(b, b)

        def compute(b, k):
            ra_b, rb_b, re_b, g_b = ras[b], rbs[b], res[b], gs[b]

            def crow(r, carry2):
                for j in range(HH // LANES):
                    sl = pl.ds(j * LANES, LANES)
                    g_b[r, sl] = _gelu_sc(ra_b[r, sl] + rb_b[r, sl]
                                          + re_b[r, sl])
                return carry2
            lax.fori_loop(0, C, crow, 0)

        def wait_gathers(b, k):
            base = pl.multiple_of(wid * EW + k * C, 8)
            pltpu.make_async_copy(a_hbm.at[dsts.at[k]], ras[b], sgs[b]).wait()
            pltpu.make_async_copy(b_hbm.at[srcs.at[k]], rbs[b], sgs[b]).wait()
            pltpu.make_async_copy(ec_hbm.at[pl.ds(base, C)], res[b],
                                  sgs[b]).wait()

        def wait_scatter(b, k):
            pltpu.make_async_copy(gs[b], s_sh.at[dsts.at[k]], sss[b]).wait()

        def step(m, carry):
            for b in range(2):
                k = 2 * m + b
                wait_gathers(b, k)

                @pl.when(m >= 1)
                def _():
                    wait_scatter(b, k)

                compute(b, k)
                pltpu.async_copy(gs[b], s_sh.at[dsts.at[k]], sss[b], add=True)

                @pl.when(k + 2 < NCHUNK)
                def _():
                    issue_gathers(k + 2, b)
            return carry
        lax.fori_loop(0, NCHUNK // 2, step, 0)

        kt = NCHUNK - 1
        wait_gathers(0, kt)
        wait_scatter(0, kt)
        compute(0, kt)
        pltpu.async_copy(gs[0], s_sh.at[dsts.at[kt]], sss[0], add=True)
        wait_scatter(1, kt)
        wait_scatter(0, kt)

        plsc.subcore_barrier()
        for t in range(ST // C):
            pltpu.sync_copy(s_sh.at[pl.ds(r0 + t * C, C)], g0)
            pltpu.sync_copy(g0, out_hbm.at[core, half].at[pl.ds(r0 + t * C, C)])
        pltpu.sync_copy(s_sh.at[pl.ds(r0 + (ST // C) * C, ST % C)],
                        g0.at[pl.ds(0, ST % C)])
        pltpu.sync_copy(g0.at[pl.ds(0, ST % C)],
                        out_hbm.at[core, half].at[pl.ds(r0 + (ST // C) * C,
                                                        ST % C)])

        @pl.when(sub == NS - 1)
        def _():
            pltpu.sync_copy(s_sh.at[pl.ds(NS * ST, TAIL)],
                            g0.at[pl.ds(0, TAIL)])
            pltpu.sync_copy(g0.at[pl.ds(0, TAIL)],
                            out_hbm.at[core, half].at[pl.ds(NS * ST, TAIL)])

        plsc.subcore_barrier()


def _edge_pass(alo, blo, eclo, ahi, bhi, echi, dsts3, srcs3):
    k = pl.kernel(
        _edge_pass_body,
        out_type=jax.ShapeDtypeStruct((NC, 2, N, HH), jnp.float32),
        mesh=_mesh(),
        compiler_params=pltpu.CompilerParams(use_tc_tiling_on_sc=False),
        scratch_types=[
            pltpu.VMEM((NCHUNK, C), jnp.int32),
            pltpu.VMEM((NCHUNK, C), jnp.int32),
            pltpu.VMEM((C, HH), jnp.float32),
            pltpu.VMEM((C, HH), jnp.float32),
            pltpu.VMEM((C, HH), jnp.float32),
            pltpu.VMEM((C, HH), jnp.float32),
            pltpu.VMEM((C, HH), jnp.float32),
            pltpu.VMEM((C, HH), jnp.float32),
            pltpu.VMEM((C, HH), jnp.float32),
            pltpu.VMEM((C, HH), jnp.float32),
            pltpu.VMEM_SHARED((N, HH), jnp.float32),
            pltpu.SemaphoreType.DMA,
            pltpu.SemaphoreType.DMA,
            pltpu.SemaphoreType.DMA,
            pltpu.SemaphoreType.DMA,
        ],
    )
    return k(alo, blo, eclo, ahi, bhi, echi, dsts3, srcs3)


def kernel(x, edge_index, edge_attr, params):
    p = params
    src = edge_index[0]
    dst = edge_index[1]

    def row(v):
        return v.reshape(1, -1)

    lys = p['layers']

    def w1ij(l):
        w1 = lys[l]['msg_W1']
        return jnp.concatenate([w1[:, :H].T, w1[:, H:2 * H].T], axis=1)

    h, alo, ahi, blo, bhi = _node_encode(
        x, p['ne_W'].T, row(p['ne_b']),
        row(p['ne_ln_g']), row(p['ne_ln_b']), w1ij(0))

    def enc(l):
        return _edge_encode(
            edge_attr, p['ee_W'].T, row(p['ee_b']),
            row(p['ee_ln_g']), row(p['ee_ln_b']),
            lys[l]['msg_W1'][:, 2 * H:].T, row(lys[l]['msg_b1']))

    eclo, echi = enc(0)

    degp = _degree(dst)
    dsts3 = dst.reshape(NW, NCHUNK, C)
    srcs3 = src.reshape(NW, NCHUNK, C)

    for l in range(2):
        lp = lys[l]
        s = _edge_pass(alo, blo, eclo, ahi, bhi, echi, dsts3, srcs3)
        eclo, echi = enc(l + 1)
        h, alo, ahi, blo, bhi = _update_pre(
            h, s, degp, lp['msg_W2'].T, row(lp['msg_b2']),
            lp['upd_W'][:, :H].T, lp['upd_W'][:, H:].T,
            row(lp['upd_b']), row(lp['upd_ln_g']), row(lp['upd_ln_b']),
            w1ij(l + 1))

    lp = lys[2]
    s = _edge_pass(alo, blo, eclo, ahi, bhi, echi, dsts3, srcs3)
    return _update_head(
        h, s, degp, lp['msg_W2'].T, row(lp['msg_b2']),
        lp['upd_W'][:, :H].T, lp['upd_W'][:, H:].T,
        row(lp['upd_b']), row(lp['upd_ln_g']), row(lp['upd_ln_b']),
        p['h_W1'].T, row(p['h_b1']), p['h_W2'].T, row(p['h_b2']),
        p['h_W3'].T, row(p['h_b3']))


# speedup vs baseline: 1.0690x; 1.0690x over previous
import jax
import jax.numpy as jnp
from jax import lax
from jax.experimental import pallas as pl
from jax.experimental.pallas import tpu as pltpu
from jax.experimental.pallas import tpu_sc as plsc

N, E = 10000, 320000
NODE_IN, EDGE_IN, H = 128, 16, 128
HH = H // 2

NC, NS, LANES = 2, 16, 16
NW = NC * NS
EW = E // NW
C = 80
NCHUNK = EW // C
ST = 624
TAIL = N - NS * ST

_SQ12 = 0.7071067811865476


def _gelu_sc(v):
    z = v * _SQ12
    az = jnp.abs(z)
    t = 1.0 / (1.0 + 0.47047 * az)
    pe = (t * (0.3480242 + t * (-0.0958798 + t * 0.7478556))) * jnp.exp(-z * z)
    q = 0.5 * v * pe
    return jnp.where(v < 0, q, v - q)


def _gelu(v):
    z = v * _SQ12
    az = jnp.abs(z)
    t = 1.0 / (1.0 + 0.3275911 * az)
    poly = t * (0.254829592 + t * (-0.284496736 + t * (1.421413741
               + t * (-1.453152027 + t * 1.061405429))))
    erf_abs = 1.0 - poly * jnp.exp(-az * az)
    erf = jnp.where(z < 0, -erf_abs, erf_abs)
    return 0.5 * v * (1.0 + erf)


def _ln(v, g, b, eps=1e-5):
    m = v.mean(-1, keepdims=True)
    var = ((v - m) ** 2).mean(-1, keepdims=True)
    return (v - m) * lax.rsqrt(var + eps) * g + b


def _dot(a, b):
    return jax.lax.dot_general(a, b, (((1,), (0,)), ((), ())),
                               preferred_element_type=jnp.float32)


_NB = 2000
_NG = N // _NB
_EB = 4000
_EG = E // _EB


def _full(shape):
    return pl.BlockSpec(shape, lambda i: tuple(0 for _ in shape))


def _node_encode_body(x_ref, w_ref, b_ref, g_ref, bb_ref, wij_ref,
                      o_ref, *ab_refs):
    y = _dot(x_ref[...], w_ref[...]) + b_ref[...]
    h = _gelu(_ln(y, g_ref[...], bb_ref[...]))
    o_ref[...] = h
    ab = _dot(h, wij_ref[...])
    for q in range(4):
        ab_refs[q][...] = ab[:, q * HH:(q + 1) * HH]


def _edge_encode_body(ea_ref, w_ref, b_ref, g_ref, bb_ref,
                      w0_ref, b0_ref, w1_ref, b1_ref, w2_ref, b2_ref,
                      *o_refs):
    y = _dot(ea_ref[...], w_ref[...]) + b_ref[...]
    ea = _gelu(_ln(y, g_ref[...], bb_ref[...]))
    for l, (wl, bl) in enumerate(((w0_ref, b0_ref), (w1_ref, b1_ref),
                                  (w2_ref, b2_ref))):
        ec = _dot(ea, wl[...]) + bl[...]
        o_refs[2 * l][...] = ec[:, :HH]
        o_refs[2 * l + 1][...] = ec[:, HH:]


def _node_pre_body(h_ref, w_ref, *o_refs):
    y = _dot(h_ref[...], w_ref[...])
    for q in range(4):
        o_refs[q][...] = y[:, q * HH:(q + 1) * HH]


def _new_h(h_ref, s_ref, dp_ref, w2_ref, b2_ref, uh_ref, ua_ref, ub_ref,
           g_ref, bb_ref):
    s2 = jnp.concatenate([s_ref[0, 0] + s_ref[1, 0],
                          s_ref[0, 1] + s_ref[1, 1]], axis=-1)
    dsum = dp_ref[0] + dp_ref[1]
    aggr = _dot(s2, w2_ref[...]) + dsum[:, 0:1] * b2_ref[...]
    t = _dot(h_ref[...], uh_ref[...]) + _dot(aggr, ua_ref[...]) + ub_ref[...]
    return h_ref[...] + _ln(t, g_ref[...], bb_ref[...])


def _update_pre_body(h_ref, s_ref, dp_ref, w2_ref, b2_ref,
                     uh_ref, ua_ref, ub_ref, g_ref, bb_ref, wij_ref,
                     o_ref, *ab_refs):
    hn = _new_h(h_ref, s_ref, dp_ref, w2_ref, b2_ref, uh_ref, ua_ref,
                ub_ref, g_ref, bb_ref)
    o_ref[...] = hn
    ab = _dot(hn, wij_ref[...])
    for q in range(4):
        ab_refs[q][...] = ab[:, q * HH:(q + 1) * HH]


def _update_head_body(h_ref, s_ref, dp_ref, w2_ref, b2_ref,
                      uh_ref, ua_ref, ub_ref, g_ref, bb_ref,
                      w1_ref, b1_ref, hw2_ref, hb2_ref, w3_ref, b3_ref,
                      o_ref, acc_ref, mut_ref):
    i = pl.program_id(0)
    hn = _new_h(h_ref, s_ref, dp_ref, w2_ref, b2_ref, uh_ref, ua_ref,
                ub_ref, g_ref, bb_ref)

    @pl.when(i == 0)
    def _():
        acc_ref[...] = jnp.zeros_like(acc_ref)
        mut_ref[...] = hn[0:1, :]

    acc_ref[...] += jnp.sum(hn, axis=0, keepdims=True)

    @pl.when(i == _NG - 1)
    def _():
        glob = acc_ref[...] * (1.0 / N)
        r = jnp.concatenate([mut_ref[...], glob], axis=-1)
        r = _gelu(_dot(r, w1_ref[...]) + b1_ref[...])
        r = _gelu(_dot(r, hw2_ref[...]) + hb2_ref[...])
        o_ref[...] = _dot(r, w3_ref[...]) + b3_ref[...]


def _head_body(h_ref, w1_ref, b1_ref, w2_ref, b2_ref, w3_ref, b3_ref,
               o_ref, acc_ref, mut_ref):
    i = pl.program_id(0)

    @pl.when(i == 0)
    def _():
        acc_ref[...] = jnp.zeros_like(acc_ref)
        mut_ref[...] = h_ref[0:1, :]

    acc_ref[...] += jnp.sum(h_ref[...], axis=0, keepdims=True)

    @pl.when(i == _NG - 1)
    def _():
        glob = acc_ref[...] * (1.0 / N)
        r = jnp.concatenate([mut_ref[...], glob], axis=-1)
        r = _gelu(_dot(r, w1_ref[...]) + b1_ref[...])
        r = _gelu(_dot(r, w2_ref[...]) + b2_ref[...])
        o_ref[...] = _dot(r, w3_ref[...]) + b3_ref[...]


def _node_encode(x, wT, b, g, bb, wijT):
    return pl.pallas_call(
        _node_encode_body,
        grid=(_NG,),
        in_specs=[pl.BlockSpec((_NB, NODE_IN), lambda i: (i, 0)),
                  _full((NODE_IN, H)), _full((1, H)), _full((1, H)),
                  _full((1, H)), _full((H, 2 * H))],
        out_specs=[pl.BlockSpec((_NB, H), lambda i: (i, 0))]
                  + [pl.BlockSpec((_NB, HH), lambda i: (i, 0))] * 4,
        out_shape=[jax.ShapeDtypeStruct((N, H), jnp.float32)]
                  + [jax.ShapeDtypeStruct((N, HH), jnp.float32)] * 4,
    )(x, wT, b, g, bb, wijT)


def _edge_encode(ea, wT, b, g, bb, w0T, b0, w1T, b1, w2T, b2):
    specs = [pl.BlockSpec((_EB, EDGE_IN), lambda i: (i, 0)),
             _full((EDGE_IN, H)), _full((1, H)), _full((1, H)), _full((1, H))]
    for _ in range(3):
        specs += [_full((H, H)), _full((1, H))]
    o_spec = pl.BlockSpec((_EB, HH), lambda i: (i, 0))
    o_shape = jax.ShapeDtypeStruct((E, HH), jnp.float32)
    return pl.pallas_call(
        _edge_encode_body,
        grid=(_EG,),
        in_specs=specs,
        out_specs=[o_spec] * 6,
        out_shape=[o_shape] * 6,
    )(ea, wT, b, g, bb, w0T, b0, w1T, b1, w2T, b2)


def _node_pre(h, wT):
    return pl.pallas_call(
        _node_pre_body,
        grid=(_NG,),
        in_specs=[pl.BlockSpec((_NB, H), lambda i: (i, 0)), _full((H, 2 * H))],
        out_specs=[pl.BlockSpec((_NB, HH), lambda i: (i, 0))] * 4,
        out_shape=[jax.ShapeDtypeStruct((N, HH), jnp.float32)] * 4,
    )(h, wT)


_UPD_SPECS = [pl.BlockSpec((_NB, H), lambda i: (i, 0)),
              pl.BlockSpec((NC, 2, _NB, HH), lambda i: (0, 0, i, 0)),
              pl.BlockSpec((NC, _NB, LANES), lambda i: (0, i, 0)),
              _full((H, H)), _full((1, H)), _full((H, H)), _full((H, H)),
              _full((1, H)), _full((1, H)), _full((1, H))]


def _update_pre(h, s, dp, w2T, b2, uhT, uaT, ub, g, bb, wijT):
    return pl.pallas_call(
        _update_pre_body,
        grid=(_NG,),
        in_specs=_UPD_SPECS + [_full((H, 2 * H))],
        out_specs=[pl.BlockSpec((_NB, H), lambda i: (i, 0))]
                  + [pl.BlockSpec((_NB, HH), lambda i: (i, 0))] * 4,
        out_shape=[jax.ShapeDtypeStruct((N, H), jnp.float32)]
                  + [jax.ShapeDtypeStruct((N, HH), jnp.float32)] * 4,
    )(h, s, dp, w2T, b2, uhT, uaT, ub, g, bb, wijT)


def _update_head(h, s, dp, w2T, b2, uhT, uaT, ub, g, bb,
                 w1T, b1, hw2T, hb2, w3T, b3):
    return pl.pallas_call(
        _update_head_body,
        grid=(_NG,),
        in_specs=_UPD_SPECS + [_full((2 * H, H)), _full((1, H)),
                               _full((H, H // 2)), _full((1, H // 2)),
                               _full((H // 2, 1)), _full((1, 1))],
        out_specs=pl.BlockSpec((1, 1), lambda i: (0, 0)),
        out_shape=jax.ShapeDtypeStruct((1, 1), jnp.float32),
        scratch_shapes=[pltpu.VMEM((1, H), jnp.float32),
                        pltpu.VMEM((1, H), jnp.float32)],
    )(h, s, dp, w2T, b2, uhT, uaT, ub, g, bb,
      w1T, b1, hw2T, hb2, w3T, b3)


def _head(h, w1T, b1, w2T, b2, w3T, b3):
    return pl.pallas_call(
        _head_body,
        grid=(_NG,),
        in_specs=[pl.BlockSpec((_NB, H), lambda i: (i, 0)),
                  _full((2 * H, H)), _full((1, H)),
                  _full((H, H // 2)), _full((1, H // 2)),
                  _full((H // 2, 1)), _full((1, 1))],
        out_specs=pl.BlockSpec((1, 1), lambda i: (0, 0)),
        out_shape=jax.ShapeDtypeStruct((1, 1), jnp.float32),
        scratch_shapes=[pltpu.VMEM((1, H), jnp.float32),
                        pltpu.VMEM((1, H), jnp.float32)],
    )(h, w1T, b1, w2T, b2, w3T, b3)


def _mesh():
    return plsc.VectorSubcoreMesh(core_axis_name="c", subcore_axis_name="s",
                                  num_cores=NC, num_subcores=NS)


_CD = 2000
_NDC = EW // _CD


def _degree_body(dst_hbm, out_hbm, idx_v, ones_v, zb_v, deg_sh, sem):
    core = lax.axis_index("c")
    sub = lax.axis_index("s")
    wid = sub * NC + core
    r0 = pl.multiple_of(sub * ST, 8)

    def fill_z(r, carry):
        zb_v[r, :] = jnp.zeros((LANES,), jnp.float32)
        return carry
    lax.fori_loop(0, ST, fill_z, 0)

    def fill_o(r, carry):
        ones_v[r, :] = jnp.ones((LANES,), jnp.float32)
        return carry
    lax.fori_loop(0, _CD, fill_o, 0)

    pltpu.sync_copy(zb_v, deg_sh.at[pl.ds(r0, ST)])

    @pl.when(sub == NS - 1)
    def _():
        pltpu.sync_copy(zb_v.at[pl.ds(0, TAIL)],
                        deg_sh.at[pl.ds(NS * ST, TAIL)])

    plsc.subcore_barrier()

    def chunk(k, carry):
        base = pl.multiple_of(wid * EW + k * _CD, 8)
        pltpu.sync_copy(dst_hbm.at[pl.ds(base, _CD)], idx_v)
        pltpu.sync_copy(ones_v, deg_sh.at[idx_v], add=True)
        return carry
    lax.fori_loop(0, _NDC, chunk, 0)

    plsc.subcore_barrier()
    pltpu.sync_copy(deg_sh.at[pl.ds(r0, ST)], zb_v)
    pltpu.sync_copy(zb_v, out_hbm.at[core].at[pl.ds(r0, ST)])

    @pl.when(sub == NS - 1)
    def _():
        pltpu.sync_copy(deg_sh.at[pl.ds(NS * ST, TAIL)],
                        zb_v.at[pl.ds(0, TAIL)])
        pltpu.sync_copy(zb_v.at[pl.ds(0, TAIL)],
                        out_hbm.at[core].at[pl.ds(NS * ST, TAIL)])


def _degree(dst):
    k = pl.kernel(
        _degree_body,
        out_type=jax.ShapeDtypeStruct((NC, N, LANES), jnp.float32),
        mesh=_mesh(),
        compiler_params=pltpu.CompilerParams(use_tc_tiling_on_sc=False),
        scratch_types=[
            pltpu.VMEM((_CD,), jnp.int32),
            pltpu.VMEM((_CD, LANES), jnp.float32),
            pltpu.VMEM((ST, LANES), jnp.float32),
            pltpu.VMEM_SHARED((N, LANES), jnp.float32),
            pltpu.SemaphoreType.DMA,
        ],
    )
    return k(dst)


def _edge_pass_body(alo_hbm, blo_hbm, eclo_hbm, ahi_hbm, bhi_hbm, echi_hbm,
                    dsts_hbm, srcs_hbm, out_hbm,
                    dsts, srcs, ra0, rb0, re0, g0, ra1, rb1, re1, g1,
                    s_sh, sg0, sg1, ss0, ss1):
    core = lax.axis_index("c")
    sub = lax.axis_index("s")
    wid = sub * NC + core
    ras = (ra0, ra1)
    rbs = (rb0, rb1)
    res = (re0, re1)
    gs = (g0, g1)
    sgs = (sg0, sg1)
    sss = (ss0, ss1)
    r0 = pl.multiple_of(sub * ST, 8)

    pltpu.sync_copy(dsts_hbm.at[wid], dsts)
    pltpu.sync_copy(srcs_hbm.at[wid], srcs)

    for half, (a_hbm, b_hbm, ec_hbm) in enumerate(
            ((alo_hbm, blo_hbm, eclo_hbm), (ahi_hbm, bhi_hbm, echi_hbm))):
        def fill_z(r, carry):
            for j in range(HH // LANES):
                g0[r, pl.ds(j * LANES, LANES)] = jnp.zeros((LANES,),
                                                           jnp.float32)
            return carry
        lax.fori_loop(0, C, fill_z, 0)
        for t in range(ST // C):
            pltpu.sync_copy(g0, s_sh.at[pl.ds(r0 + t * C, C)])
        pltpu.sync_copy(g0.at[pl.ds(0, ST % C)],
                        s_sh.at[pl.ds(r0 + (ST // C) * C, ST % C)])

        @pl.when(sub == NS - 1)
        def _():
            pltpu.sync_copy(g0.at[pl.ds(0, TAIL)],
                            s_sh.at[pl.ds(NS * ST, TAIL)])

        plsc.subcore_barrier()

        def issue_gathers(kk, b):
            base = pl.multiple_of(wid * EW + kk * C, 8)
            pltpu.async_copy(a_hbm.at[dsts.at[kk]], ras[b], sgs[b])
            pltpu.async_copy(b_hbm.at[srcs.at[kk]], rbs[b], sgs[b])
            pltpu.async_copy(ec_hbm.at[pl.ds(base, C)], res[b], sgs[b])

        for b in range(2):
            issue_gathers(b, b)

        def compute(b, k):
            ra_b, rb_b, re_b, g_b = ras[b], rbs[b], res[b], gs[b]

            def crow(i, carry2):
                r = pl.multiple_of(i * 2, 2)
                for u in range(2):
                    for j in range(HH // LANES):
                        sl = pl.ds(j * LANES, LANES)
                        g_b[r + u, sl] = _gelu_sc(ra_b[r + u, sl]
                                                  + rb_b[r + u, sl]
                                                  + re_b[r + u, sl])
                return carry2
            lax.fori_loop(0, C // 2, crow, 0)

        def wait_gathers(b, k):
            base = pl.multiple_of(wid * EW + k * C, 8)
            pltpu.make_async_copy(a_hbm.at[dsts.at[k]], ras[b], sgs[b]).wait()
            pltpu.make_async_copy(b_hbm.at[srcs.at[k]], rbs[b], sgs[b]).wait()
            pltpu.make_async_copy(ec_hbm.at[pl.ds(base, C)], res[b],
                                  sgs[b]).wait()

        def wait_scatter(b, k):
            pltpu.make_async_copy(gs[b], s_sh.at[dsts.at[k]], sss[b]).wait()

        def step(m, carry):
            for b in range(2):
                k = 2 * m + b
                wait_gathers(b, k)

                @pl.when(m >= 1)
                def _():
                    wait_scatter(b, k)

                compute(b, k)
                pltpu.async_copy(gs[b], s_sh.at[dsts.at[k]], sss[b], add=True)

                @pl.when(k + 2 < NCHUNK)
                def _():
                    issue_gathers(k + 2, b)
            return carry
        lax.fori_loop(0, NCHUNK // 2, step, 0)

        kt = NCHUNK - 1
        wait_gathers(0, kt)
        wait_scatter(0, kt)
        compute(0, kt)
        pltpu.async_copy(gs[0], s_sh.at[dsts.at[kt]], sss[0], add=True)
        wait_scatter(1, kt)
        wait_scatter(0, kt)

        plsc.subcore_barrier()
        for t in range(ST // C):
            pltpu.sync_copy(s_sh.at[pl.ds(r0 + t * C, C)], g0)
            pltpu.sync_copy(g0, out_hbm.at[core, half].at[pl.ds(r0 + t * C, C)])
        pltpu.sync_copy(s_sh.at[pl.ds(r0 + (ST // C) * C, ST % C)],
                        g0.at[pl.ds(0, ST % C)])
        pltpu.sync_copy(g0.at[pl.ds(0, ST % C)],
                        out_hbm.at[core, half].at[pl.ds(r0 + (ST // C) * C,
                                                        ST % C)])

        @pl.when(sub == NS - 1)
        def _():
            pltpu.sync_copy(s_sh.at[pl.ds(NS * ST, TAIL)],
                            g0.at[pl.ds(0, TAIL)])
            pltpu.sync_copy(g0.at[pl.ds(0, TAIL)],
                            out_hbm.at[core, half].at[pl.ds(NS * ST, TAIL)])

        plsc.subcore_barrier()


def _edge_pass(alo, blo, eclo, ahi, bhi, echi, dsts3, srcs3):
    k = pl.kernel(
        _edge_pass_body,
        out_type=jax.ShapeDtypeStruct((NC, 2, N, HH), jnp.float32),
        mesh=_mesh(),
        compiler_params=pltpu.CompilerParams(use_tc_tiling_on_sc=False),
        scratch_types=[
            pltpu.VMEM((NCHUNK, C), jnp.int32),
            pltpu.VMEM((NCHUNK, C), jnp.int32),
            pltpu.VMEM((C, HH), jnp.float32),
            pltpu.VMEM((C, HH), jnp.float32),
            pltpu.VMEM((C, HH), jnp.float32),
            pltpu.VMEM((C, HH), jnp.float32),
            pltpu.VMEM((C, HH), jnp.float32),
            pltpu.VMEM((C, HH), jnp.float32),
            pltpu.VMEM((C, HH), jnp.float32),
            pltpu.VMEM((C, HH), jnp.float32),
            pltpu.VMEM_SHARED((N, HH), jnp.float32),
            pltpu.SemaphoreType.DMA,
            pltpu.SemaphoreType.DMA,
            pltpu.SemaphoreType.DMA,
            pltpu.SemaphoreType.DMA,
        ],
    )
    return k(alo, blo, eclo, ahi, bhi, echi, dsts3, srcs3)


def kernel(x, edge_index, edge_attr, params):
    p = params
    src = edge_index[0]
    dst = edge_index[1]

    def row(v):
        return v.reshape(1, -1)

    lys = p['layers']

    def w1ij(l):
        w1 = lys[l]['msg_W1']
        return jnp.concatenate([w1[:, :H].T, w1[:, H:2 * H].T], axis=1)

    h, alo, ahi, blo, bhi = _node_encode(
        x, p['ne_W'].T, row(p['ne_b']),
        row(p['ne_ln_g']), row(p['ne_ln_b']), w1ij(0))

    ecs = _edge_encode(
        edge_attr, p['ee_W'].T, row(p['ee_b']),
        row(p['ee_ln_g']), row(p['ee_ln_b']),
        lys[0]['msg_W1'][:, 2 * H:].T, row(lys[0]['msg_b1']),
        lys[1]['msg_W1'][:, 2 * H:].T, row(lys[1]['msg_b1']),
        lys[2]['msg_W1'][:, 2 * H:].T, row(lys[2]['msg_b1']))

    degp = _degree(dst)
    dsts3 = dst.reshape(NW, NCHUNK, C)
    srcs3 = src.reshape(NW, NCHUNK, C)

    for l in range(2):
        lp = lys[l]
        s = _edge_pass(alo, blo, ecs[2 * l], ahi, bhi, ecs[2 * l + 1],
                       dsts3, srcs3)
        h, alo, ahi, blo, bhi = _update_pre(
            h, s, degp, lp['msg_W2'].T, row(lp['msg_b2']),
            lp['upd_W'][:, :H].T, lp['upd_W'][:, H:].T,
            row(lp['upd_b']), row(lp['upd_ln_g']), row(lp['upd_ln_b']),
            w1ij(l + 1))

    lp = lys[2]
    s = _edge_pass(alo, blo, ecs[4], ahi, bhi, ecs[5], dsts3, srcs3)
    return _update_head(
        h, s, degp, lp['msg_W2'].T, row(lp['msg_b2']),
        lp['upd_W'][:, :H].T, lp['upd_W'][:, H:].T,
        row(lp['upd_b']), row(lp['upd_ln_g']), row(lp['upd_ln_b']),
        p['h_W1'].T, row(p['h_b1']), p['h_W2'].T, row(p['h_b2']),
        p['h_W3'].T, row(p['h_b3']))


# speedup vs baseline: 1.0775x; 1.0080x over previous
import jax
import jax.numpy as jnp
from jax import lax
from jax.experimental import pallas as pl
from jax.experimental.pallas import tpu as pltpu
from jax.experimental.pallas import tpu_sc as plsc

N, E = 10000, 320000
NODE_IN, EDGE_IN, H = 128, 16, 128
HH = H // 2

NC, NS, LANES = 2, 16, 16
NW = NC * NS
EW = E // NW
C = 80
NCHUNK = EW // C
ST = 624
TAIL = N - NS * ST

_SQ12 = 0.7071067811865476


def _gelu_sc(v):
    z = v * _SQ12
    az = jnp.abs(z)
    t = 1.0 / (1.0 + 0.47047 * az)
    pe = (t * (0.3480242 + t * (-0.0958798 + t * 0.7478556))) * jnp.exp(-z * z)
    q = 0.5 * v * pe
    return jnp.where(v < 0, q, v - q)


def _gelu(v):
    z = v * _SQ12
    az = jnp.abs(z)
    t = 1.0 / (1.0 + 0.3275911 * az)
    poly = t * (0.254829592 + t * (-0.284496736 + t * (1.421413741
               + t * (-1.453152027 + t * 1.061405429))))
    erf_abs = 1.0 - poly * jnp.exp(-az * az)
    erf = jnp.where(z < 0, -erf_abs, erf_abs)
    return 0.5 * v * (1.0 + erf)


def _ln(v, g, b, eps=1e-5):
    m = v.mean(-1, keepdims=True)
    var = ((v - m) ** 2).mean(-1, keepdims=True)
    return (v - m) * lax.rsqrt(var + eps) * g + b


def _dot(a, b):
    return jax.lax.dot_general(a, b, (((1,), (0,)), ((), ())),
                               preferred_element_type=jnp.float32)


_NB = 2000
_NG = N // _NB
_EB = 4000
_EG = E // _EB


def _full(shape):
    return pl.BlockSpec(shape, lambda i: tuple(0 for _ in shape))


def _node_encode_body(x_ref, w_ref, b_ref, g_ref, bb_ref, wij_ref,
                      o_ref, *ab_refs):
    y = _dot(x_ref[...], w_ref[...]) + b_ref[...]
    h = _gelu(_ln(y, g_ref[...], bb_ref[...]))
    o_ref[...] = h
    ab = _dot(h, wij_ref[...])
    for q in range(4):
        ab_refs[q][...] = ab[:, q * HH:(q + 1) * HH]


def _edge_encode_body(ea_ref, w_ref, b_ref, g_ref, bb_ref,
                      w0_ref, b0_ref, w1_ref, b1_ref, w2_ref, b2_ref,
                      *o_refs):
    y = _dot(ea_ref[...], w_ref[...]) + b_ref[...]
    ea = _gelu(_ln(y, g_ref[...], bb_ref[...]))
    for l, (wl, bl) in enumerate(((w0_ref, b0_ref), (w1_ref, b1_ref),
                                  (w2_ref, b2_ref))):
        ec = _dot(ea, wl[...]) + bl[...]
        o_refs[2 * l][...] = ec[:, :HH]
        o_refs[2 * l + 1][...] = ec[:, HH:]


def _node_pre_body(h_ref, w_ref, *o_refs):
    y = _dot(h_ref[...], w_ref[...])
    for q in range(4):
        o_refs[q][...] = y[:, q * HH:(q + 1) * HH]


def _new_h(h_ref, s_ref, dp_ref, w2_ref, b2_ref, uh_ref, ua_ref, ub_ref,
           g_ref, bb_ref):
    s2 = jnp.concatenate([s_ref[0, 0] + s_ref[1, 0],
                          s_ref[0, 1] + s_ref[1, 1]], axis=-1)
    dsum = dp_ref[0] + dp_ref[1]
    aggr = _dot(s2, w2_ref[...]) + dsum[:, 0:1] * b2_ref[...]
    t = _dot(h_ref[...], uh_ref[...]) + _dot(aggr, ua_ref[...]) + ub_ref[...]
    return h_ref[...] + _ln(t, g_ref[...], bb_ref[...])


def _update_pre_body(h_ref, s_ref, dp_ref, w2_ref, b2_ref,
                     uh_ref, ua_ref, ub_ref, g_ref, bb_ref, wij_ref,
                     o_ref, *ab_refs):
    hn = _new_h(h_ref, s_ref, dp_ref, w2_ref, b2_ref, uh_ref, ua_ref,
                ub_ref, g_ref, bb_ref)
    o_ref[...] = hn
    ab = _dot(hn, wij_ref[...])
    for q in range(4):
        ab_refs[q][...] = ab[:, q * HH:(q + 1) * HH]


def _update_head_body(h_ref, s_ref, dp_ref, w2_ref, b2_ref,
                      uh_ref, ua_ref, ub_ref, g_ref, bb_ref,
                      w1_ref, b1_ref, hw2_ref, hb2_ref, w3_ref, b3_ref,
                      o_ref, acc_ref, mut_ref):
    i = pl.program_id(0)
    hn = _new_h(h_ref, s_ref, dp_ref, w2_ref, b2_ref, uh_ref, ua_ref,
                ub_ref, g_ref, bb_ref)

    @pl.when(i == 0)
    def _():
        acc_ref[...] = jnp.zeros_like(acc_ref)
        mut_ref[...] = hn[0:1, :]

    acc_ref[...] += jnp.sum(hn, axis=0, keepdims=True)

    @pl.when(i == _NG - 1)
    def _():
        glob = acc_ref[...] * (1.0 / N)
        r = jnp.concatenate([mut_ref[...], glob], axis=-1)
        r = _gelu(_dot(r, w1_ref[...]) + b1_ref[...])
        r = _gelu(_dot(r, hw2_ref[...]) + hb2_ref[...])
        o_ref[...] = _dot(r, w3_ref[...]) + b3_ref[...]


def _head_body(h_ref, w1_ref, b1_ref, w2_ref, b2_ref, w3_ref, b3_ref,
               o_ref, acc_ref, mut_ref):
    i = pl.program_id(0)

    @pl.when(i == 0)
    def _():
        acc_ref[...] = jnp.zeros_like(acc_ref)
        mut_ref[...] = h_ref[0:1, :]

    acc_ref[...] += jnp.sum(h_ref[...], axis=0, keepdims=True)

    @pl.when(i == _NG - 1)
    def _():
        glob = acc_ref[...] * (1.0 / N)
        r = jnp.concatenate([mut_ref[...], glob], axis=-1)
        r = _gelu(_dot(r, w1_ref[...]) + b1_ref[...])
        r = _gelu(_dot(r, w2_ref[...]) + b2_ref[...])
        o_ref[...] = _dot(r, w3_ref[...]) + b3_ref[...]


def _node_encode(x, wT, b, g, bb, wijT):
    return pl.pallas_call(
        _node_encode_body,
        grid=(_NG,),
        in_specs=[pl.BlockSpec((_NB, NODE_IN), lambda i: (i, 0)),
                  _full((NODE_IN, H)), _full((1, H)), _full((1, H)),
                  _full((1, H)), _full((H, 2 * H))],
        out_specs=[pl.BlockSpec((_NB, H), lambda i: (i, 0))]
                  + [pl.BlockSpec((_NB, HH), lambda i: (i, 0))] * 4,
        out_shape=[jax.ShapeDtypeStruct((N, H), jnp.float32)]
                  + [jax.ShapeDtypeStruct((N, HH), jnp.float32)] * 4,
    )(x, wT, b, g, bb, wijT)


def _edge_encode(ea, wT, b, g, bb, w0T, b0, w1T, b1, w2T, b2):
    specs = [pl.BlockSpec((_EB, EDGE_IN), lambda i: (i, 0)),
             _full((EDGE_IN, H)), _full((1, H)), _full((1, H)), _full((1, H))]
    for _ in range(3):
        specs += [_full((H, H)), _full((1, H))]
    o_spec = pl.BlockSpec((_EB, HH), lambda i: (i, 0))
    o_shape = jax.ShapeDtypeStruct((E, HH), jnp.float32)
    return pl.pallas_call(
        _edge_encode_body,
        grid=(_EG,),
        in_specs=specs,
        out_specs=[o_spec] * 6,
        out_shape=[o_shape] * 6,
    )(ea, wT, b, g, bb, w0T, b0, w1T, b1, w2T, b2)


def _node_pre(h, wT):
    return pl.pallas_call(
        _node_pre_body,
        grid=(_NG,),
        in_specs=[pl.BlockSpec((_NB, H), lambda i: (i, 0)), _full((H, 2 * H))],
        out_specs=[pl.BlockSpec((_NB, HH), lambda i: (i, 0))] * 4,
        out_shape=[jax.ShapeDtypeStruct((N, HH), jnp.float32)] * 4,
    )(h, wT)


_UPD_SPECS = [pl.BlockSpec((_NB, H), lambda i: (i, 0)),
              pl.BlockSpec((NC, 2, _NB, HH), lambda i: (0, 0, i, 0)),
              pl.BlockSpec((NC, _NB, LANES), lambda i: (0, i, 0)),
              _full((H, H)), _full((1, H)), _full((H, H)), _full((H, H)),
              _full((1, H)), _full((1, H)), _full((1, H))]


def _update_pre(h, s, dp, w2T, b2, uhT, uaT, ub, g, bb, wijT):
    return pl.pallas_call(
        _update_pre_body,
        grid=(_NG,),
        in_specs=_UPD_SPECS + [_full((H, 2 * H))],
        out_specs=[pl.BlockSpec((_NB, H), lambda i: (i, 0))]
                  + [pl.BlockSpec((_NB, HH), lambda i: (i, 0))] * 4,
        out_shape=[jax.ShapeDtypeStruct((N, H), jnp.float32)]
                  + [jax.ShapeDtypeStruct((N, HH), jnp.float32)] * 4,
    )(h, s, dp, w2T, b2, uhT, uaT, ub, g, bb, wijT)


def _update_head(h, s, dp, w2T, b2, uhT, uaT, ub, g, bb,
                 w1T, b1, hw2T, hb2, w3T, b3):
    return pl.pallas_call(
        _update_head_body,
        grid=(_NG,),
        in_specs=_UPD_SPECS + [_full((2 * H, H)), _full((1, H)),
                               _full((H, H // 2)), _full((1, H // 2)),
                               _full((H // 2, 1)), _full((1, 1))],
        out_specs=pl.BlockSpec((1, 1), lambda i: (0, 0)),
        out_shape=jax.ShapeDtypeStruct((1, 1), jnp.float32),
        scratch_shapes=[pltpu.VMEM((1, H), jnp.float32),
                        pltpu.VMEM((1, H), jnp.float32)],
    )(h, s, dp, w2T, b2, uhT, uaT, ub, g, bb,
      w1T, b1, hw2T, hb2, w3T, b3)


def _head(h, w1T, b1, w2T, b2, w3T, b3):
    return pl.pallas_call(
        _head_body,
        grid=(_NG,),
        in_specs=[pl.BlockSpec((_NB, H), lambda i: (i, 0)),
                  _full((2 * H, H)), _full((1, H)),
                  _full((H, H // 2)), _full((1, H // 2)),
                  _full((H // 2, 1)), _full((1, 1))],
        out_specs=pl.BlockSpec((1, 1), lambda i: (0, 0)),
        out_shape=jax.ShapeDtypeStruct((1, 1), jnp.float32),
        scratch_shapes=[pltpu.VMEM((1, H), jnp.float32),
                        pltpu.VMEM((1, H), jnp.float32)],
    )(h, w1T, b1, w2T, b2, w3T, b3)


def _mesh():
    return plsc.VectorSubcoreMesh(core_axis_name="c", subcore_axis_name="s",
                                  num_cores=NC, num_subcores=NS)


_CD = 2000
_NDC = EW // _CD


def _degree_body(dst_hbm, out_hbm, idx_v, ones_v, zb_v, deg_sh, sem):
    core = lax.axis_index("c")
    sub = lax.axis_index("s")
    wid = sub * NC + core
    r0 = pl.multiple_of(sub * ST, 8)

    def fill_z(r, carry):
        zb_v[r, :] = jnp.zeros((LANES,), jnp.float32)
        return carry
    lax.fori_loop(0, ST, fill_z, 0)

    def fill_o(r, carry):
        ones_v[r, :] = jnp.ones((LANES,), jnp.float32)
        return carry
    lax.fori_loop(0, _CD, fill_o, 0)

    pltpu.sync_copy(zb_v, deg_sh.at[pl.ds(r0, ST)])

    @pl.when(sub == NS - 1)
    def _():
        pltpu.sync_copy(zb_v.at[pl.ds(0, TAIL)],
                        deg_sh.at[pl.ds(NS * ST, TAIL)])

    plsc.subcore_barrier()

    def chunk(k, carry):
        base = pl.multiple_of(wid * EW + k * _CD, 8)
        pltpu.sync_copy(dst_hbm.at[pl.ds(base, _CD)], idx_v)
        pltpu.sync_copy(ones_v, deg_sh.at[idx_v], add=True)
        return carry
    lax.fori_loop(0, _NDC, chunk, 0)

    plsc.subcore_barrier()
    pltpu.sync_copy(deg_sh.at[pl.ds(r0, ST)], zb_v)
    pltpu.sync_copy(zb_v, out_hbm.at[core].at[pl.ds(r0, ST)])

    @pl.when(sub == NS - 1)
    def _():
        pltpu.sync_copy(deg_sh.at[pl.ds(NS * ST, TAIL)],
                        zb_v.at[pl.ds(0, TAIL)])
        pltpu.sync_copy(zb_v.at[pl.ds(0, TAIL)],
                        out_hbm.at[core].at[pl.ds(NS * ST, TAIL)])


def _degree(dst):
    k = pl.kernel(
        _degree_body,
        out_type=jax.ShapeDtypeStruct((NC, N, LANES), jnp.float32),
        mesh=_mesh(),
        compiler_params=pltpu.CompilerParams(use_tc_tiling_on_sc=False),
        scratch_types=[
            pltpu.VMEM((_CD,), jnp.int32),
            pltpu.VMEM((_CD, LANES), jnp.float32),
            pltpu.VMEM((ST, LANES), jnp.float32),
            pltpu.VMEM_SHARED((N, LANES), jnp.float32),
            pltpu.SemaphoreType.DMA,
        ],
    )
    return k(dst)


def _edge_pass_body(alo_hbm, blo_hbm, eclo_hbm, ahi_hbm, bhi_hbm, echi_hbm,
                    dsts_hbm, srcs_hbm, out_hbm,
                    dsts, srcs, ra0, rb0, re0, g0, ra1, rb1, re1, g1,
                    s_sh, sg0, sg1, ss0, ss1):
    core = lax.axis_index("c")
    sub = lax.axis_index("s")
    wid = sub * NC + core
    ras = (ra0, ra1)
    rbs = (rb0, rb1)
    res = (re0, re1)
    gs = (g0, g1)
    sgs = (sg0, sg1)
    sss = (ss0, ss1)
    r0 = pl.multiple_of(sub * ST, 8)

    pltpu.sync_copy(dsts_hbm.at[wid], dsts)
    pltpu.sync_copy(srcs_hbm.at[wid], srcs)

    for half, (a_hbm, b_hbm, ec_hbm) in enumerate(
            ((alo_hbm, blo_hbm, eclo_hbm), (ahi_hbm, bhi_hbm, echi_hbm))):
        def fill_z(r, carry):
            for j in range(HH // LANES):
                g0[r, pl.ds(j * LANES, LANES)] = jnp.zeros((LANES,),
                                                           jnp.float32)
            return carry
        lax.fori_loop(0, C, fill_z, 0)
        for t in range(ST // C):
            pltpu.sync_copy(g0, s_sh.at[pl.ds(r0 + t * C, C)])
        pltpu.sync_copy(g0.at[pl.ds(0, ST % C)],
                        s_sh.at[pl.ds(r0 + (ST // C) * C, ST % C)])

        @pl.when(sub == NS - 1)
        def _():
            pltpu.sync_copy(g0.at[pl.ds(0, TAIL)],
                            s_sh.at[pl.ds(NS * ST, TAIL)])

        plsc.subcore_barrier()

        def issue_gathers(kk, b):
            base = pl.multiple_of(wid * EW + kk * C, 8)
            pltpu.async_copy(a_hbm.at[dsts.at[kk]], ras[b], sgs[b])
            pltpu.async_copy(b_hbm.at[srcs.at[kk]], rbs[b], sgs[b])
            pltpu.async_copy(ec_hbm.at[pl.ds(base, C)], res[b], sgs[b])

        for b in range(2):
            issue_gathers(b, b)

        def compute(b, k):
            ra_b, rb_b, re_b, g_b = ras[b], rbs[b], res[b], gs[b]

            def crow(r, carry2):
                for j in range(HH // LANES):
                    sl = pl.ds(j * LANES, LANES)
                    g_b[r, sl] = _gelu_sc(ra_b[r, sl] + rb_b[r, sl]
                                          + re_b[r, sl])
                return carry2
            lax.fori_loop(0, C, crow, 0)

        def wait_gathers(b, k):
            base = pl.multiple_of(wid * EW + k * C, 8)
            pltpu.make_async_copy(a_hbm.at[dsts.at[k]], ras[b], sgs[b]).wait()
            pltpu.make_async_copy(b_hbm.at[srcs.at[k]], rbs[b], sgs[b]).wait()
            pltpu.make_async_copy(ec_hbm.at[pl.ds(base, C)], res[b],
                                  sgs[b]).wait()

        def wait_scatter(b, k):
            pltpu.make_async_copy(gs[b], s_sh.at[dsts.at[k]], sss[b]).wait()

        def step(m, carry):
            for b in range(2):
                k = 2 * m + b
                wait_gathers(b, k)

                @pl.when(m >= 1)
                def _():
                    wait_scatter(b, k)

                compute(b, k)
                pltpu.async_copy(gs[b], s_sh.at[dsts.at[k]], sss[b], add=True)

                @pl.when(k + 2 < NCHUNK)
                def _():
                    issue_gathers(k + 2, b)
            return carry
        lax.fori_loop(0, NCHUNK // 2, step, 0)

        kt = NCHUNK - 1
        wait_gathers(0, kt)
        wait_scatter(0, kt)
        compute(0, kt)
        pltpu.async_copy(gs[0], s_sh.at[dsts.at[kt]], sss[0], add=True)
        wait_scatter(1, kt)
        wait_scatter(0, kt)

        plsc.subcore_barrier()
        for t in range(ST // C):
            pltpu.sync_copy(s_sh.at[pl.ds(r0 + t * C, C)], g0)
            pltpu.sync_copy(g0, out_hbm.at[core, half].at[pl.ds(r0 + t * C, C)])
        pltpu.sync_copy(s_sh.at[pl.ds(r0 + (ST // C) * C, ST % C)],
                        g0.at[pl.ds(0, ST % C)])
        pltpu.sync_copy(g0.at[pl.ds(0, ST % C)],
                        out_hbm.at[core, half].at[pl.ds(r0 + (ST // C) * C,
                                                        ST % C)])

        @pl.when(sub == NS - 1)
        def _():
            pltpu.sync_copy(s_sh.at[pl.ds(NS * ST, TAIL)],
                            g0.at[pl.ds(0, TAIL)])
            pltpu.sync_copy(g0.at[pl.ds(0, TAIL)],
                            out_hbm.at[core, half].at[pl.ds(NS * ST, TAIL)])

        plsc.subcore_barrier()


def _edge_pass(alo, blo, eclo, ahi, bhi, echi, dsts3, srcs3):
    k = pl.kernel(
        _edge_pass_body,
        out_type=jax.ShapeDtypeStruct((NC, 2, N, HH), jnp.float32),
        mesh=_mesh(),
        compiler_params=pltpu.CompilerParams(use_tc_tiling_on_sc=False),
        scratch_types=[
            pltpu.VMEM((NCHUNK, C), jnp.int32),
            pltpu.VMEM((NCHUNK, C), jnp.int32),
            pltpu.VMEM((C, HH), jnp.float32),
            pltpu.VMEM((C, HH), jnp.float32),
            pltpu.VMEM((C, HH), jnp.float32),
            pltpu.VMEM((C, HH), jnp.float32),
            pltpu.VMEM((C, HH), jnp.float32),
            pltpu.VMEM((C, HH), jnp.float32),
            pltpu.VMEM((C, HH), jnp.float32),
            pltpu.VMEM((C, HH), jnp.float32),
            pltpu.VMEM_SHARED((N, HH), jnp.float32),
            pltpu.SemaphoreType.DMA,
            pltpu.SemaphoreType.DMA,
            pltpu.SemaphoreType.DMA,
            pltpu.SemaphoreType.DMA,
        ],
    )
    return k(alo, blo, eclo, ahi, bhi, echi, dsts3, srcs3)


def kernel(x, edge_index, edge_attr, params):
    p = params
    src = edge_index[0]
    dst = edge_index[1]

    def row(v):
        return v.reshape(1, -1)

    lys = p['layers']

    def w1ij(l):
        w1 = lys[l]['msg_W1']
        return jnp.concatenate([w1[:, :H].T, w1[:, H:2 * H].T], axis=1)

    h, alo, ahi, blo, bhi = _node_encode(
        x, p['ne_W'].T, row(p['ne_b']),
        row(p['ne_ln_g']), row(p['ne_ln_b']), w1ij(0))

    ecs = _edge_encode(
        edge_attr, p['ee_W'].T, row(p['ee_b']),
        row(p['ee_ln_g']), row(p['ee_ln_b']),
        lys[0]['msg_W1'][:, 2 * H:].T, row(lys[0]['msg_b1']),
        lys[1]['msg_W1'][:, 2 * H:].T, row(lys[1]['msg_b1']),
        lys[2]['msg_W1'][:, 2 * H:].T, row(lys[2]['msg_b1']))

    degp = _degree(dst)
    dsts3 = dst.reshape(NW, NCHUNK, C)
    srcs3 = src.reshape(NW, NCHUNK, C)

    for l in range(2):
        lp = lys[l]
        s = _edge_pass(alo, blo, ecs[2 * l], ahi, bhi, ecs[2 * l + 1],
                       dsts3, srcs3)
        h, alo, ahi, blo, bhi = _update_pre(
            h, s, degp, lp['msg_W2'].T, row(lp['msg_b2']),
            lp['upd_W'][:, :H].T, lp['upd_W'][:, H:].T,
            row(lp['upd_b']), row(lp['upd_ln_g']), row(lp['upd_ln_b']),
            w1ij(l + 1))

    lp = lys[2]
    s = _edge_pass(alo, blo, ecs[4], ahi, bhi, ecs[5], dsts3, srcs3)
    return _update_head(
        h, s, degp, lp['msg_W2'].T, row(lp['msg_b2']),
        lp['upd_W'][:, :H].T, lp['upd_W'][:, H:].T,
        row(lp['upd_b']), row(lp['upd_ln_g']), row(lp['upd_ln_b']),
        p['h_W1'].T, row(p['h_b1']), p['h_W2'].T, row(p['h_b2']),
        p['h_W3'].T, row(p['h_b3']))
